# baseline (speedup 1.0000x reference)
import functools
import jax
import jax.numpy as jnp
from jax import lax
from jax.experimental import pallas as pl
from jax.experimental.pallas import tpu as pltpu
from jax.experimental.pallas import tpu_sc as plsc

_B, _S, _D = 4, 8192, 1024
_CH = 8
_CHF = _CH * _D
_NW = 32
_ROWS_PER_W = _S // _NW
_NCHUNK = _ROWS_PER_W // _CH


def _sc_add(x, pos):
    mesh = plsc.VectorSubcoreMesh(core_axis_name="c", subcore_axis_name="s")

    @functools.partial(
        pl.kernel,
        mesh=mesh,
        out_type=jax.ShapeDtypeStruct((_B, _S, _D), jnp.float32),
        compiler_params=pltpu.CompilerParams(use_tc_tiling_on_sc=True),
        scratch_types=[
            pltpu.VMEM((2, _B, _CH, _D), jnp.float32),
            pltpu.VMEM((2, _CH, _D), jnp.float32),
            pltpu.SemaphoreType.DMA((2,)),
            pltpu.SemaphoreType.DMA((2,)),
            pltpu.SemaphoreType.DMA((2,)),
        ],
    )
    def body(x_hbm, pos_hbm, out_hbm, xb, pb, sxin, sout, spos):
        wid = lax.axis_index("s") * 2 + lax.axis_index("c")
        row0 = wid * _ROWS_PER_W

        def x_in(c, p):
            src = x_hbm.at[:, pl.ds(row0 + c * _CH, _CH)]
            return pltpu.make_async_copy(src, xb.at[p], sxin.at[p])

        def x_out(c, p):
            dst = out_hbm.at[:, pl.ds(row0 + c * _CH, _CH)]
            return pltpu.make_async_copy(xb.at[p], dst, sout.at[p])

        def pos_in(c, p):
            src = pos_hbm.at[pl.ds(row0 + c * _CH, _CH)]
            return pltpu.make_async_copy(src, pb.at[p], spos.at[p])

        pos_in(0, 0).start()
        x_in(0, 0).start()

        def chunk_pair(cc, carry):
            for p in range(2):
                c = cc * 2 + p

                @pl.when(c > 0)
                def _():
                    x_out(c - 1, 1 - p).wait()

                @pl.when(c + 1 < _NCHUNK)
                def _():
                    pos_in(c + 1, 1 - p).start()
                    x_in(c + 1, 1 - p).start()

                pos_in(c, p).wait()
                x_in(c, p).wait()

                for b in range(_B):
                    @plsc.parallel_loop(0, _CHF, step=16, unroll=8)
                    def _(i):
                        r = lax.shift_right_logical(i, 10)
                        col = pl.multiple_of(lax.bitwise_and(i, _D - 1), 16)
                        sl = pl.ds(col, 16)
                        plsc.addupdate(xb.at[p, b, r, sl], pb[p, r, sl])

                x_out(c, p).start()
            return carry

        lax.fori_loop(0, _NCHUNK // 2, chunk_pair, 0)

        x_out(_NCHUNK - 1, (_NCHUNK - 1) % 2).wait()

    return body(x, pos)


def kernel(x, position_embeddings):
    return _sc_add(x, position_embeddings[: x.shape[1]])


# speedup vs baseline: 1.1229x; 1.1229x over previous
import functools
import jax
import jax.numpy as jnp
from jax import lax
from jax.experimental import pallas as pl
from jax.experimental.pallas import tpu as pltpu
from jax.experimental.pallas import tpu_sc as plsc

_B, _S, _D = 4, 8192, 1024
_CH = 16
_CHF = _CH * _D
_NW = 32
_ROWS_PER_W = _S // _NW
_NCHUNK = _ROWS_PER_W // _CH
_NBUF = 4


def _sc_add(x, pos):
    mesh = plsc.VectorSubcoreMesh(core_axis_name="c", subcore_axis_name="s")

    @functools.partial(
        pl.kernel,
        mesh=mesh,
        out_type=jax.ShapeDtypeStruct((_B, _S, _D), jnp.float32),
        compiler_params=pltpu.CompilerParams(use_tc_tiling_on_sc=True),
        scratch_types=[
            pltpu.VMEM((_NBUF, _CH, _D), jnp.float32),
            pltpu.VMEM((2, _CH, _D), jnp.float32),
            pltpu.SemaphoreType.DMA((_NBUF,)),
            pltpu.SemaphoreType.DMA((_NBUF,)),
            pltpu.SemaphoreType.DMA((2,)),
        ],
    )
    def body(x_hbm, pos_hbm, out_hbm, xb, pb, sxin, sout, spos):
        wid = lax.axis_index("s") * 2 + lax.axis_index("c")
        row0 = wid * _ROWS_PER_W

        def x_in(c, b, p):
            src = x_hbm.at[b, pl.ds(row0 + c * _CH, _CH)]
            return pltpu.make_async_copy(src, xb.at[p], sxin.at[p])

        def x_out(c, b, p):
            dst = out_hbm.at[b, pl.ds(row0 + c * _CH, _CH)]
            return pltpu.make_async_copy(xb.at[p], dst, sout.at[p])

        def pos_in(c, q):
            src = pos_hbm.at[pl.ds(row0 + c * _CH, _CH)]
            return pltpu.make_async_copy(src, pb.at[q], spos.at[q])

        pos_in(0, 0).start()
        x_in(0, 0, 0).start()
        x_in(0, 1, 1).start()

        def chunk(k, carry):
            q = lax.bitwise_and(k, 1)
            for j in range(_B):
                if j < 2:
                    @pl.when(k >= 1)
                    def _():
                        x_out(k - 1, j + 2, j + 2).wait()
                    x_in(k, j + 2, j + 2).start()
                else:
                    @pl.when(k + 1 < _NCHUNK)
                    def _():
                        x_out(k, j - 2, j - 2).wait()
                        x_in(k + 1, j - 2, j - 2).start()

                if j == 0:
                    @pl.when(k + 1 < _NCHUNK)
                    def _():
                        pos_in(k + 1, 1 - q).start()
                    pos_in(k, q).wait()

                x_in(k, j, j).wait()

                @plsc.parallel_loop(0, _CHF, step=16, unroll=8)
                def _(i):
                    r = lax.shift_right_logical(i, 10)
                    col = pl.multiple_of(lax.bitwise_and(i, _D - 1), 16)
                    sl = pl.ds(col, 16)
                    plsc.addupdate(xb.at[j, r, sl], pb[q, r, sl])

                x_out(k, j, j).start()
            return carry

        lax.fori_loop(0, _NCHUNK, chunk, 0)

        for j in range(_B):
            x_out(_NCHUNK - 1, j, j).wait()

    return body(x, pos)


def kernel(x, position_embeddings):
    return _sc_add(x, position_embeddings[: x.shape[1]])
